# Initial kernel scaffold; baseline (speedup 1.0000x reference)
import jax
import jax.numpy as jnp
from jax.experimental import pallas as pl


def kernel(x, edge_index, edge_weight, batch, W_lin, b_lin, W1, b1, W2, b2):
    raise NotImplementedError("write your pallas kernel here")


# speedup vs baseline: 4.9122x; 4.9122x over previous
import functools

import jax
import jax.numpy as jnp
from jax import lax
from jax.experimental import pallas as pl
from jax.experimental.pallas import tpu as pltpu
from jax.experimental.pallas import tpu_sc as plsc

N_GRAPHS = 64
NC, NS, L = 2, 16, 16
NW = NC * NS
C = 80
RPT = 624
SPANS = [(t * 80, 80) for t in range(7)] + [(560, 64)]
PR = 320


def _lin_body(x_ref, w_ref, b_ref, o_ref):
    o_ref[...] = (
        jnp.dot(x_ref[...], w_ref[...], preferred_element_type=jnp.float32)
        + b_ref[...]
    )


def _linear(x, w_t, b, blk):
    n, d_in = x.shape
    d_out = w_t.shape[1]
    return pl.pallas_call(
        _lin_body,
        grid=(n // blk,),
        in_specs=[
            pl.BlockSpec((blk, d_in), lambda i: (i, 0)),
            pl.BlockSpec((d_in, d_out), lambda i: (0, 0)),
            pl.BlockSpec((1, d_out), lambda i: (0, 0)),
        ],
        out_specs=pl.BlockSpec((blk, d_out), lambda i: (i, 0)),
        out_shape=jax.ShapeDtypeStruct((n, d_out), jnp.float32),
    )(x, w_t, b.reshape(1, -1))


def _make_edge_kernel(n, d, nchunk):
    nj = d // L
    rpt = RPT
    tail = n - NS * rpt
    mesh = plsc.VectorSubcoreMesh(core_axis_name="c", subcore_axis_name="s")

    def body(h_hbm, src_hbm, dst_hbm, w_hbm, out_hbm,
             aggr, sidx_all, didx_all, sidx, didx, wv, rows,
             sem):
        cid = lax.axis_index("c")
        sid = lax.axis_index("s")
        wid = cid * NS + sid

        def zrow(r, carry):
            for j in range(nj):
                rows[r, pl.ds(j * L, L)] = jnp.zeros((L,), jnp.float32)
            return carry
        lax.fori_loop(0, C, zrow, 0)
        zbase = sid * rpt
        for off, sz in SPANS:
            pltpu.sync_copy(rows.at[pl.ds(0, sz)],
                            aggr.at[pl.ds(zbase + off, sz)])

        @pl.when(sid == NS - 1)
        def _zero_tail():
            pltpu.sync_copy(rows.at[pl.ds(0, tail)],
                            aggr.at[pl.ds(NS * rpt, tail)])

        pltpu.sync_copy(src_hbm.at[wid], sidx_all)
        pltpu.sync_copy(dst_hbm.at[wid], didx_all)
        plsc.subcore_barrier()

        def chunk(k, carry):
            pltpu.sync_copy(w_hbm.at[wid, k], wv)
            for i in range(C // L):
                sl = pl.ds(i * L, L)
                sidx[sl] = sidx_all[k, sl]
                didx[sl] = didx_all[k, sl]
            pltpu.async_copy(h_hbm.at[sidx], rows, sem).wait()
            def scale(g, cc):
                w16 = wv[pl.ds(g * L, L)]
                for i in range(L):
                    wt = w16[i]
                    ei = g * L + i
                    for j in range(nj):
                        sl = pl.ds(j * L, L)
                        rows[ei, sl] = rows[ei, sl] * wt
                return cc
            lax.fori_loop(0, C // L, scale, 0)
            pltpu.sync_copy(rows, aggr.at[didx], add=True)
            return carry
        lax.fori_loop(0, nchunk, chunk, 0)

        plsc.subcore_barrier()
        for off, sz in SPANS:
            sl = pl.ds(zbase + off, sz)
            pltpu.sync_copy(aggr.at[sl], rows.at[pl.ds(0, sz)])
            pltpu.sync_copy(rows.at[pl.ds(0, sz)], out_hbm.at[cid, sl])

        @pl.when(sid == NS - 1)
        def _read_tail():
            sl = pl.ds(NS * rpt, tail)
            pltpu.sync_copy(aggr.at[sl], rows.at[pl.ds(0, tail)])
            pltpu.sync_copy(rows.at[pl.ds(0, tail)], out_hbm.at[cid, sl])

    return pl.kernel(
        body,
        out_type=jax.ShapeDtypeStruct((NC, n, d), jnp.float32),
        mesh=mesh,
        scratch_types=[
            pltpu.VMEM_SHARED((n, d), jnp.float32),
            pltpu.VMEM((nchunk, C), jnp.int32),
            pltpu.VMEM((nchunk, C), jnp.int32),
            pltpu.VMEM((C,), jnp.int32),
            pltpu.VMEM((C,), jnp.int32),
            pltpu.VMEM((C,), jnp.float32),
            pltpu.VMEM((C, d), jnp.float32),
            pltpu.SemaphoreType.DMA,
        ],
    )


def _mid_body(p_ref, w_ref, b_ref, o_ref):
    h = jnp.maximum(p_ref[0] + p_ref[1], 0.0)
    o_ref[...] = (
        jnp.dot(h, w_ref[...], preferred_element_type=jnp.float32)
        + b_ref[...]
    )


def _mid(partials, w1_t, b1, blk):
    _, n, d = partials.shape
    d_out = w1_t.shape[1]
    return pl.pallas_call(
        _mid_body,
        grid=(n // blk,),
        in_specs=[
            pl.BlockSpec((2, blk, d), lambda i: (0, i, 0)),
            pl.BlockSpec((d, d_out), lambda i: (0, 0)),
            pl.BlockSpec((1, d_out), lambda i: (0, 0)),
        ],
        out_specs=pl.BlockSpec((blk, d_out), lambda i: (i, 0)),
        out_shape=jax.ShapeDtypeStruct((n, d_out), jnp.float32),
    )(partials, w1_t, b1.reshape(1, -1))


def _make_pool_kernel(n, d):
    nj = d // L
    last_base = n - PR
    mesh = plsc.VectorSubcoreMesh(core_axis_name="c", subcore_axis_name="s")

    def body(h_hbm, b_hbm, out_hbm, hbuf, ids, acc, sem):
        cid = lax.axis_index("c")
        sid = lax.axis_index("s")
        wid = cid * NS + sid
        base = jnp.where(wid == NW - 1, last_base, wid * PR)
        base = pl.multiple_of(base, 8)

        def arow(g, carry):
            for j in range(nj):
                acc[g, pl.ds(j * L, L)] = jnp.full((L,), -jnp.inf,
                                                   jnp.float32)
            return carry
        lax.fori_loop(0, N_GRAPHS, arow, 0)

        pltpu.sync_copy(h_hbm.at[pl.ds(base, PR)], hbuf)
        pltpu.sync_copy(b_hbm.at[pl.ds(base, PR)], ids)

        def grp(gi, carry):
            ids16 = ids[pl.ds(gi * L, L)]
            for i in range(L):
                g = ids16[i]
                r = gi * L + i
                for j in range(nj):
                    sl = pl.ds(j * L, L)
                    acc[g, sl] = jnp.maximum(acc[g, sl], hbuf[r, sl])
            return carry
        lax.fori_loop(0, PR // L, grp, 0)

        pltpu.sync_copy(acc, out_hbm.at[wid])

    return pl.kernel(
        body,
        out_type=jax.ShapeDtypeStruct((NW, N_GRAPHS, d), jnp.float32),
        mesh=mesh,
        scratch_types=[
            pltpu.VMEM((PR, d), jnp.float32),
            pltpu.VMEM((PR,), jnp.int32),
            pltpu.VMEM((N_GRAPHS, d), jnp.float32),
            pltpu.SemaphoreType.DMA,
        ],
    )


def _fin_body(p_ref, w_ref, b_ref, o_ref):
    pooled = jnp.max(p_ref[...], axis=0)
    o_ref[...] = (
        jnp.dot(pooled, w_ref[...], preferred_element_type=jnp.float32)
        + b_ref[...]
    )


def _fin(pool_partials, w2_t, b2):
    d_out = w2_t.shape[1]
    return pl.pallas_call(
        _fin_body,
        out_shape=jax.ShapeDtypeStruct((N_GRAPHS, d_out), jnp.float32),
    )(pool_partials, w2_t, b2.reshape(1, -1))


def kernel(x, edge_index, edge_weight, batch, W_lin, b_lin, W1, b1, W2, b2):
    n, _ = x.shape
    e = edge_index.shape[1]
    d = W_lin.shape[0]
    epw = e // NW
    nchunk = epw // C

    src = edge_index[0].astype(jnp.int32).reshape(NW, nchunk, C)
    dst = edge_index[1].astype(jnp.int32).reshape(NW, nchunk, C)
    w = edge_weight.astype(jnp.float32).reshape(NW, nchunk, C)
    bids = batch.astype(jnp.int32)

    h = _linear(x, W_lin.T, b_lin, blk=1000)
    partials = _make_edge_kernel(n, d, nchunk)(h, src, dst, w)
    h2 = _mid(partials, W1.T, b1, blk=1000)
    pool_partials = _make_pool_kernel(n, d)(h2, bids)
    return _fin(pool_partials, W2.T, b2)
